# baseline (speedup 1.0000x reference)
import functools

import jax
import jax.numpy as jnp
from jax import lax
from jax.experimental import pallas as pl
from jax.experimental.pallas import tpu as pltpu
from jax.experimental.pallas import tpu_sc as plsc

N, E, D, G = 10000, 320000, 128, 16
NC, NS = 2, 16
NW = NC * NS
NT = 10240
NSR = 10112
NCNT = NSR // 2
EPW = 10032
EPAD = NW * EPW
CHUNK = 24
NCHUNK = EPW // CHUNK
TW = 2 * D
ROWS_PT = NSR // NS
TROWS = 104
DLT = 1.0 / 15.0
PCLAMP = 99.995


def _tc_pre(hpad, pos16, w1a, w1b, be1):
    blk = 256

    def body(h_ref, p_ref, wa_ref, wb_ref, b_ref, ta_ref, tb_ref):
        hb = h_ref[...]
        a = jnp.dot(hb, wa_ref[...], preferred_element_type=jnp.float32)
        b = jnp.dot(hb, wb_ref[...], preferred_element_type=jnp.float32)
        p = p_ref[...]
        z = jnp.zeros((blk, TW - D - 16), jnp.float32)
        ta_ref[...] = jnp.concatenate([a, p, z], axis=1)
        tb_ref[...] = jnp.concatenate([b + b_ref[...], p, z], axis=1)

    return pl.pallas_call(
        body,
        grid=(NT // blk,),
        in_specs=[
            pl.BlockSpec((blk, D), lambda i: (i, 0)),
            pl.BlockSpec((blk, 16), lambda i: (i, 0)),
            pl.BlockSpec((D, D), lambda i: (0, 0)),
            pl.BlockSpec((D, D), lambda i: (0, 0)),
            pl.BlockSpec((1, D), lambda i: (0, 0)),
        ],
        out_specs=[
            pl.BlockSpec((blk, TW), lambda i: (i, 0)),
            pl.BlockSpec((blk, TW), lambda i: (i, 0)),
        ],
        out_shape=[
            jax.ShapeDtypeStruct((NT, TW), jnp.float32),
            jax.ShapeDtypeStruct((NT, TW), jnp.float32),
        ],
    )(hpad, pos16, w1a, w1b, be1)


def _tc_tbl(offsw, invw, wr):
    def body(o_ref, iw_ref, wr_ref, t_ref):
        di = (lax.broadcasted_iota(jnp.int32, (TROWS, G), 0)
              .astype(jnp.float32) - 1.0) * DLT
        t = di * iw_ref[...] - o_ref[...]
        feats = jnp.exp(-0.5 * t * t)
        t_ref[...] = jnp.dot(feats, wr_ref[...],
                             preferred_element_type=jnp.float32)

    full = lambda shape: pl.BlockSpec(shape, lambda: (0, 0))
    return pl.pallas_call(
        body,
        in_specs=[full((1, G)), full((1, G)), full((G, D))],
        out_specs=full((TROWS, D)),
        out_shape=jax.ShapeDtypeStruct((TROWS, D), jnp.float32),
    )(offsw, invw, wr)


def _sc_edge(ta, tb, srcp, dstp, tblf, zrows):
    mesh = plsc.VectorSubcoreMesh(core_axis_name="c", subcore_axis_name="s")
    f32 = jnp.float32

    @functools.partial(
        pl.kernel,
        mesh=mesh,
        compiler_params=pltpu.CompilerParams(needs_layout_passes=False),
        out_type=[
            jax.ShapeDtypeStruct((NC, NSR, D), jnp.float32),
            jax.ShapeDtypeStruct((NW * NCNT,), jnp.int32),
        ],
        scratch_types=[
            pltpu.VMEM((2, CHUNK), jnp.int32),
            pltpu.VMEM((2, CHUNK), jnp.int32),
            pltpu.VMEM((2, CHUNK), jnp.int32),
            pltpu.VMEM((2, CHUNK, TW), jnp.float32),
            pltpu.VMEM((2, CHUNK, TW), jnp.float32),
            pltpu.VMEM((2, CHUNK, D), jnp.float32),
            pltpu.VMEM((TROWS * D,), jnp.float32),
            pltpu.VMEM((NCNT,), jnp.int32),
            pltpu.VMEM_SHARED((NSR, D), jnp.float32),
            pltpu.SemaphoreType.DMA((2,)),
            pltpu.SemaphoreType.DMA((2,)),
            pltpu.SemaphoreType.DMA((2,)),
        ],
    )
    def k(ta_h, tb_h, src_h, dst_h, tbl_h, z_h, out_h, outc_h,
          sidx, didx, dsc, ga, gb, vbuf, tbl_v, cnt_v, s_sh, sa, sb, ss):
        cid = lax.axis_index("c")
        sid = lax.axis_index("s")
        wid = sid * NC + cid

        pltpu.sync_copy(tbl_h, tbl_v)

        def zc(i, _):
            cnt_v[pl.ds(16 * i, 16)] = jnp.zeros((16,), jnp.int32)
            return 0

        lax.fori_loop(0, NCNT // 16, zc, 0)

        pltpu.sync_copy(z_h, s_sh.at[pl.ds(sid * ROWS_PT, ROWS_PT)])
        plsc.subcore_barrier()

        lanes = lax.broadcasted_iota(jnp.int32, (16,), 0)
        one_i = jnp.full((16,), 1, jnp.int32)
        hi8 = lanes >= 8
        ebase = wid * EPW

        def fetch(j, p):
            pltpu.sync_copy(src_h.at[pl.ds(ebase + j * CHUNK, CHUNK)],
                            sidx.at[p])
            pltpu.sync_copy(dst_h.at[pl.ds(ebase + j * CHUNK, CHUNK)],
                            didx.at[p])
            pltpu.async_copy(ta_h.at[sidx.at[p]], ga.at[p], sa.at[p])
            pltpu.async_copy(tb_h.at[didx.at[p]], gb.at[p], sb.at[p])

        def wait_gathers(p):
            pltpu.make_async_copy(ta_h.at[sidx.at[p]], ga.at[p],
                                  sa.at[p]).wait()
            pltpu.make_async_copy(tb_h.at[didx.at[p]], gb.at[p],
                                  sb.at[p]).wait()

        def wait_scatter(p):
            pltpu.make_async_copy(vbuf.at[p], s_sh.at[dsc.at[p]],
                                  ss.at[p]).wait()

        def half(t2, p):
            def prefetch():
                fetch(2 * t2 + p + 1, 1 - p)

            if p == 0:
                prefetch()
            else:
                pl.when(t2 < NCHUNK // 2 - 1)(prefetch)

            wait_gathers(p)
            pl.when(t2 >= 1)(lambda: wait_scatter(p))

            dw = didx[p, pl.ds(0, 16)]
            plsc.addupdate_scatter(
                cnt_v, [lax.shift_right_logical(dw, jnp.int32(1))],
                one_i + (dw & 1) * 65535)
            dsc[p, pl.ds(0, 16)] = dw
            dw2 = didx[p, pl.ds(8, 16)]
            plsc.addupdate_scatter(
                cnt_v, [lax.shift_right_logical(dw2, jnp.int32(1))],
                one_i + (dw2 & 1) * 65535, mask=hi8)
            plsc.store_scatter(dsc, [jnp.full((16,), p, jnp.int32),
                                     8 + lanes], dw2, mask=hi8)

            @plsc.parallel_loop(0, CHUNK, 1, unroll=4)
            def edge_body(e):
                dv = ga[p, e, pl.ds(D, 16)] - gb[p, e, pl.ds(D, 16)]
                s2 = dv * dv
                for sh in (8, 4, 2, 1):
                    s2 = s2 + s2.at[lanes ^ sh].get(
                        mode="promise_in_bounds")
                s2e = jnp.maximum(s2, 1e-30)
                ib = lax.bitcast_convert_type(s2e, jnp.int32)
                ib = 0x5F3759DF - lax.shift_right_logical(ib, jnp.int32(1))
                y = lax.bitcast_convert_type(ib, jnp.float32)
                y = y * (1.5 - 0.5 * s2e * y * y)
                y = y * (1.5 - 0.5 * s2e * y * y)
                y = y * (1.5 - 0.5 * s2e * y * y)
                dist = s2 * y
                pp = jnp.minimum(dist * (1.0 / DLT), PCLAMP)
                iv = pp.astype(jnp.int32)
                t = pp - iv.astype(f32)
                tm1 = t - 1.0
                tm2 = t - 2.0
                tp1 = t + 1.0
                a = tm1 * tm2
                b = tp1 * t
                w0 = a * t * (-1.0 / 6.0)
                w1 = a * tp1 * 0.5
                w2 = b * tm2 * (-0.5)
                w3 = b * tm1 * (1.0 / 6.0)
                ibase = iv * D + lanes
                for q in range(8):
                    i0 = ibase + 16 * q
                    t0 = plsc.load_gather(tbl_v, [i0])
                    t1 = plsc.load_gather(tbl_v, [i0 + D])
                    t2x = plsc.load_gather(tbl_v, [i0 + 2 * D])
                    t3 = plsc.load_gather(tbl_v, [i0 + 3 * D])
                    u = (ga[p, e, pl.ds(16 * q, 16)]
                         + gb[p, e, pl.ds(16 * q, 16)]
                         + (w0 * t0 + w1 * t1 + w2 * t2x + w3 * t3))
                    vbuf[p, e, pl.ds(16 * q, 16)] = u / (1.0 + jnp.exp(-u))

            pltpu.async_copy(vbuf.at[p], s_sh.at[dsc.at[p]], ss.at[p],
                             add=True)

        fetch(0, 0)

        def pair_body(t2, _):
            half(t2, 0)
            half(t2, 1)
            return 0

        lax.fori_loop(0, NCHUNK // 2, pair_body, 0)
        wait_scatter(0)
        wait_scatter(1)

        pltpu.sync_copy(cnt_v, outc_h.at[pl.ds(wid * NCNT, NCNT)])
        plsc.subcore_barrier()
        pltpu.sync_copy(
            s_sh.at[pl.ds(sid * ROWS_PT, ROWS_PT)],
            out_h.at[cid, pl.ds(sid * ROWS_PT, ROWS_PT)],
        )

    return k(ta, tb, srcp, dstp, tblf, zrows)


def _tc_post(h, s0, s1, cntT, we2, be2, wn1, bn1, wn2, bn2, wsc, bsc,
             gamma, beta):
    blk = 400

    def body(h_ref, s0_ref, s1_ref, c_ref, we2_ref, be2_ref, wn1_ref,
             bn1_ref, wn2_ref, bn2_ref, wsc_ref, bsc_ref, g_ref, be_ref,
             o_ref):
        s = s0_ref[...] + s1_ref[...]
        cnt = jnp.sum(c_ref[...], axis=1, keepdims=True)
        agg = (jnp.dot(s, we2_ref[...], preferred_element_type=jnp.float32)
               + cnt * be2_ref[...])
        x = jnp.concatenate([h_ref[...], agg], axis=-1)
        t1 = jnp.dot(x, wn1_ref[...], preferred_element_type=jnp.float32)
        t1 = t1 + bn1_ref[...]
        t1 = t1 * (1.0 / (1.0 + jnp.exp(-t1)))
        y = (jnp.dot(t1, wn2_ref[...], preferred_element_type=jnp.float32)
             + bn2_ref[...]
             + jnp.dot(x, wsc_ref[...], preferred_element_type=jnp.float32)
             + bsc_ref[...])
        mu = jnp.mean(y, axis=-1, keepdims=True)
        yc = y - mu
        var = jnp.mean(yc * yc, axis=-1, keepdims=True)
        o_ref[...] = yc * lax.rsqrt(var + 1e-5) * g_ref[...] + be_ref[...]

    full = lambda shape: pl.BlockSpec(shape, lambda i: (0, 0))
    return pl.pallas_call(
        body,
        grid=(N // blk,),
        in_specs=[
            pl.BlockSpec((blk, D), lambda i: (i, 0)),
            pl.BlockSpec((blk, D), lambda i: (i, 0)),
            pl.BlockSpec((blk, D), lambda i: (i, 0)),
            pl.BlockSpec((blk, NW), lambda i: (i, 0)),
            full((D, D)),
            full((1, D)),
            full((2 * D, D)),
            full((1, D)),
            full((D, D)),
            full((1, D)),
            full((2 * D, D)),
            full((1, D)),
            full((1, D)),
            full((1, D)),
        ],
        out_specs=pl.BlockSpec((blk, D), lambda i: (i, 0)),
        out_shape=jax.ShapeDtypeStruct((N, D), jnp.float32),
    )(h, s0, s1, cntT, we2, be2, wn1, bn1, wn2, bn2, wsc, bsc, gamma, beta)


def kernel(h, pos, edge_index, W_e1, b_e1, W_e2, b_e2, W_n1, b_n1, W_n2,
           b_n2, W_sc, b_sc, gamma, beta, offsets):
    f32 = jnp.float32
    w1a = W_e1[:D]
    w1b = W_e1[D:2 * D]
    wr = W_e1[2 * D:]
    width = offsets[1] - offsets[0]
    offsw = (offsets / width).astype(f32).reshape(1, G)
    invw = jnp.full((1, G), 1.0, f32) / width

    hpad = jnp.zeros((NT, D), f32).at[:N].set(h)
    pos16 = jnp.zeros((NT, 16), f32).at[:N, :3].set(pos)
    ta, tb = _tc_pre(hpad, pos16, w1a, w1b, b_e1.reshape(1, D))
    tbl = _tc_tbl(offsw, invw, wr)

    pad_idx = jnp.full((EPAD - E,), N, jnp.int32)
    srcp = jnp.concatenate([edge_index[0], pad_idx])
    dstp = jnp.concatenate([edge_index[1], pad_idx])
    zrows = jnp.zeros((ROWS_PT, D), f32)
    part, outc = _sc_edge(ta, tb, srcp, dstp, tbl.reshape(-1), zrows)

    c2 = outc.reshape(NW, NCNT)
    counts = jnp.stack([c2 & 0xFFFF, lax.shift_right_logical(c2, 16)],
                       axis=-1).reshape(NW, NSR)
    cntT = counts.T[:N].astype(f32)
    h_new = _tc_post(h, part[0, :N], part[1, :N], cntT, W_e2,
                     b_e2.reshape(1, D), W_n1, b_n1.reshape(1, D), W_n2,
                     b_n2.reshape(1, D), W_sc, b_sc.reshape(1, D),
                     gamma.reshape(1, D), beta.reshape(1, D))
    return (h_new, pos)


# speedup vs baseline: 1.4497x; 1.4497x over previous
import functools

import jax
import jax.numpy as jnp
from jax import lax
from jax.experimental import pallas as pl
from jax.experimental.pallas import tpu as pltpu
from jax.experimental.pallas import tpu_sc as plsc

N, E, D, G = 10000, 320000, 128, 16
NC, NS = 2, 16
NW = NC * NS
NT = 10240
NSR = 10112
NCNT = NSR // 2
EPW = 10240
EPAD = NW * EPW
CHUNK = 32
NCHUNK = EPW // CHUNK
TW = 2 * D
ROWS_PT = NSR // NS


def _tc_pre(hpad, pos16, w1a, w1b, be1):
    blk = 256

    def body(h_ref, p_ref, wa_ref, wb_ref, b_ref, ta_ref, tb_ref):
        hb = h_ref[...]
        a = jnp.dot(hb, wa_ref[...], preferred_element_type=jnp.float32)
        b = jnp.dot(hb, wb_ref[...], preferred_element_type=jnp.float32)
        p = p_ref[...]
        z = jnp.zeros((blk, TW - D - 16), jnp.float32)
        ta_ref[...] = jnp.concatenate([a, p, z], axis=1)
        tb_ref[...] = jnp.concatenate([b + b_ref[...], p, z], axis=1)

    return pl.pallas_call(
        body,
        grid=(NT // blk,),
        in_specs=[
            pl.BlockSpec((blk, D), lambda i: (i, 0)),
            pl.BlockSpec((blk, 16), lambda i: (i, 0)),
            pl.BlockSpec((D, D), lambda i: (0, 0)),
            pl.BlockSpec((D, D), lambda i: (0, 0)),
            pl.BlockSpec((1, D), lambda i: (0, 0)),
        ],
        out_specs=[
            pl.BlockSpec((blk, TW), lambda i: (i, 0)),
            pl.BlockSpec((blk, TW), lambda i: (i, 0)),
        ],
        out_shape=[
            jax.ShapeDtypeStruct((NT, TW), jnp.float32),
            jax.ShapeDtypeStruct((NT, TW), jnp.float32),
        ],
    )(hpad, pos16, w1a, w1b, be1)


def _sc_edge(ta, tb, srcp, dstp, wr, offsw, invw, zrows):
    mesh = plsc.VectorSubcoreMesh(core_axis_name="c", subcore_axis_name="s")
    f32 = jnp.float32

    @functools.partial(
        pl.kernel,
        mesh=mesh,
        compiler_params=pltpu.CompilerParams(needs_layout_passes=False),
        out_type=[
            jax.ShapeDtypeStruct((NC, NSR, D), jnp.float32),
            jax.ShapeDtypeStruct((NW * NCNT,), jnp.int32),
        ],
        scratch_types=[
            pltpu.VMEM((2, CHUNK), jnp.int32),
            pltpu.VMEM((2, CHUNK), jnp.int32),
            pltpu.VMEM((2, CHUNK), jnp.int32),
            pltpu.VMEM((2, CHUNK, TW), jnp.float32),
            pltpu.VMEM((2, CHUNK, TW), jnp.float32),
            pltpu.VMEM((2, CHUNK, D), jnp.float32),
            pltpu.VMEM((G, D), jnp.float32),
            pltpu.VMEM((16,), jnp.float32),
            pltpu.VMEM((16,), jnp.float32),
            pltpu.VMEM((NCNT,), jnp.int32),
            pltpu.VMEM_SHARED((NSR, D), jnp.float32),
            pltpu.SemaphoreType.DMA((2,)),
            pltpu.SemaphoreType.DMA((2,)),
            pltpu.SemaphoreType.DMA((2,)),
        ],
    )
    def k(ta_h, tb_h, src_h, dst_h, wr_h, offsw_h, invw_h, z_h,
          out_h, outc_h,
          sidx, didx, dsc, ga, gb, vbuf, wr_v, offs_v, invw_v, cnt_v, s_sh,
          sa, sb, ss):
        cid = lax.axis_index("c")
        sid = lax.axis_index("s")
        wid = sid * NC + cid

        pltpu.sync_copy(wr_h, wr_v)
        pltpu.sync_copy(offsw_h, offs_v)
        pltpu.sync_copy(invw_h, invw_v)

        def zc(i, _):
            cnt_v[pl.ds(16 * i, 16)] = jnp.zeros((16,), jnp.int32)
            return 0

        lax.fori_loop(0, NCNT // 16, zc, 0)

        pltpu.sync_copy(z_h, s_sh.at[pl.ds(sid * ROWS_PT, ROWS_PT)])
        plsc.subcore_barrier()

        lanes = lax.broadcasted_iota(jnp.int32, (16,), 0)
        one_i = jnp.full((16,), 1, jnp.int32)
        ebase = wid * EPW

        def fetch(j, p):
            pltpu.sync_copy(src_h.at[pl.ds(ebase + j * CHUNK, CHUNK)],
                            sidx.at[p])
            pltpu.sync_copy(dst_h.at[pl.ds(ebase + j * CHUNK, CHUNK)],
                            didx.at[p])
            pltpu.async_copy(ta_h.at[sidx.at[p]], ga.at[p], sa.at[p])
            pltpu.async_copy(tb_h.at[didx.at[p]], gb.at[p], sb.at[p])

        def wait_gathers(p):
            pltpu.make_async_copy(ta_h.at[sidx.at[p]], ga.at[p],
                                  sa.at[p]).wait()
            pltpu.make_async_copy(tb_h.at[didx.at[p]], gb.at[p],
                                  sb.at[p]).wait()

        def wait_scatter(p):
            pltpu.make_async_copy(vbuf.at[p], s_sh.at[dsc.at[p]],
                                  ss.at[p]).wait()

        def half(t2, p):
            j = 2 * t2 + p

            def prefetch():
                fetch(j + 1, 1 - p)

            if p == 0:
                prefetch()
            else:
                pl.when(t2 < NCHUNK // 2 - 1)(prefetch)

            wait_gathers(p)
            pl.when(t2 >= 1)(lambda: wait_scatter(p))

            for g2 in range(CHUNK // 16):
                dw = didx[p, pl.ds(16 * g2, 16)]
                idx2 = lax.shift_right_logical(dw, jnp.int32(1))
                val = one_i + (dw & 1) * 65535
                plsc.addupdate_scatter(cnt_v, [idx2], val)
                dsc[p, pl.ds(16 * g2, 16)] = dw

            def edge_body(e, __):
                dv = ga[p, e, pl.ds(D, 16)] - gb[p, e, pl.ds(D, 16)]
                s2 = dv * dv
                for sh in (8, 4, 2, 1):
                    s2 = s2 + s2.at[lanes ^ sh].get(
                        mode="promise_in_bounds")
                s2e = jnp.maximum(s2, 1e-30)
                ib = lax.bitcast_convert_type(s2e, jnp.int32)
                ib = 0x5F3759DF - lax.shift_right_logical(ib, jnp.int32(1))
                y = lax.bitcast_convert_type(ib, jnp.float32)
                y = y * (1.5 - 0.5 * s2e * y * y)
                y = y * (1.5 - 0.5 * s2e * y * y)
                y = y * (1.5 - 0.5 * s2e * y * y)
                dist = s2 * y
                t = dist * invw_v[...] - offs_v[...]
                f = jnp.exp(-0.5 * t * t)
                us = [ga[p, e, pl.ds(16 * q, 16)]
                      + gb[p, e, pl.ds(16 * q, 16)] for q in range(8)]
                for g in range(G):
                    fg = f.at[jnp.full((16,), g, jnp.int32)].get(
                        mode="promise_in_bounds")
                    for q in range(8):
                        us[q] = us[q] + fg * wr_v[g, pl.ds(16 * q, 16)]
                for q in range(8):
                    u = us[q]
                    vbuf[p, e, pl.ds(16 * q, 16)] = u / (1.0 + jnp.exp(-u))
                return 0

            lax.fori_loop(0, CHUNK, edge_body, 0, unroll=2)
            pltpu.async_copy(vbuf.at[p], s_sh.at[dsc.at[p]], ss.at[p],
                             add=True)

        fetch(0, 0)

        def pair_body(t2, _):
            half(t2, 0)
            half(t2, 1)
            return 0

        lax.fori_loop(0, NCHUNK // 2, pair_body, 0)
        wait_scatter(0)
        wait_scatter(1)

        pltpu.sync_copy(cnt_v, outc_h.at[pl.ds(wid * NCNT, NCNT)])
        plsc.subcore_barrier()
        pltpu.sync_copy(
            s_sh.at[pl.ds(sid * ROWS_PT, ROWS_PT)],
            out_h.at[cid, pl.ds(sid * ROWS_PT, ROWS_PT)],
        )

    return k(ta, tb, srcp, dstp, wr, offsw, invw, zrows)


def _tc_post(h, s0, s1, cntT, we2, be2, wn1, bn1, wn2, bn2, wsc, bsc,
             gamma, beta):
    blk = 400

    def body(h_ref, s0_ref, s1_ref, c_ref, we2_ref, be2_ref, wn1_ref,
             bn1_ref, wn2_ref, bn2_ref, wsc_ref, bsc_ref, g_ref, be_ref,
             o_ref):
        s = s0_ref[...] + s1_ref[...]
        cnt = jnp.sum(c_ref[...], axis=1, keepdims=True)
        agg = (jnp.dot(s, we2_ref[...], preferred_element_type=jnp.float32)
               + cnt * be2_ref[...])
        x = jnp.concatenate([h_ref[...], agg], axis=-1)
        t1 = jnp.dot(x, wn1_ref[...], preferred_element_type=jnp.float32)
        t1 = t1 + bn1_ref[...]
        t1 = t1 * (1.0 / (1.0 + jnp.exp(-t1)))
        y = (jnp.dot(t1, wn2_ref[...], preferred_element_type=jnp.float32)
             + bn2_ref[...]
             + jnp.dot(x, wsc_ref[...], preferred_element_type=jnp.float32)
             + bsc_ref[...])
        mu = jnp.mean(y, axis=-1, keepdims=True)
        yc = y - mu
        var = jnp.mean(yc * yc, axis=-1, keepdims=True)
        o_ref[...] = yc * lax.rsqrt(var + 1e-5) * g_ref[...] + be_ref[...]

    full = lambda shape: pl.BlockSpec(shape, lambda i: (0, 0))
    return pl.pallas_call(
        body,
        grid=(N // blk,),
        in_specs=[
            pl.BlockSpec((blk, D), lambda i: (i, 0)),
            pl.BlockSpec((blk, D), lambda i: (i, 0)),
            pl.BlockSpec((blk, D), lambda i: (i, 0)),
            pl.BlockSpec((blk, NW), lambda i: (i, 0)),
            full((D, D)),
            full((1, D)),
            full((2 * D, D)),
            full((1, D)),
            full((D, D)),
            full((1, D)),
            full((2 * D, D)),
            full((1, D)),
            full((1, D)),
            full((1, D)),
        ],
        out_specs=pl.BlockSpec((blk, D), lambda i: (i, 0)),
        out_shape=jax.ShapeDtypeStruct((N, D), jnp.float32),
    )(h, s0, s1, cntT, we2, be2, wn1, bn1, wn2, bn2, wsc, bsc, gamma, beta)


def kernel(h, pos, edge_index, W_e1, b_e1, W_e2, b_e2, W_n1, b_n1, W_n2,
           b_n2, W_sc, b_sc, gamma, beta, offsets):
    f32 = jnp.float32
    w1a = W_e1[:D]
    w1b = W_e1[D:2 * D]
    wr = W_e1[2 * D:]
    width = offsets[1] - offsets[0]
    offsw = (offsets / width).astype(f32)
    invw = jnp.full((16,), 1.0, f32) / width

    hpad = jnp.zeros((NT, D), f32).at[:N].set(h)
    pos16 = jnp.zeros((NT, 16), f32).at[:N, :3].set(pos)
    ta, tb = _tc_pre(hpad, pos16, w1a, w1b, b_e1.reshape(1, D))

    pad_idx = jnp.full((EPAD - E,), N, jnp.int32)
    srcp = jnp.concatenate([edge_index[0], pad_idx])
    dstp = jnp.concatenate([edge_index[1], pad_idx])
    zrows = jnp.zeros((ROWS_PT, D), f32)
    part, outc = _sc_edge(ta, tb, srcp, dstp, wr, offsw, invw, zrows)

    c2 = outc.reshape(NW, NCNT)
    counts = jnp.stack([c2 & 0xFFFF, lax.shift_right_logical(c2, 16)],
                       axis=-1).reshape(NW, NSR)
    cntT = counts.T[:N].astype(f32)
    h_new = _tc_post(h, part[0, :N], part[1, :N], cntT, W_e2,
                     b_e2.reshape(1, D), W_n1, b_n1.reshape(1, D), W_n2,
                     b_n2.reshape(1, D), W_sc, b_sc.reshape(1, D),
                     gamma.reshape(1, D), beta.reshape(1, D))
    return (h_new, pos)


# speedup vs baseline: 1.5472x; 1.0672x over previous
import functools

import jax
import jax.numpy as jnp
from jax import lax
from jax.experimental import pallas as pl
from jax.experimental.pallas import tpu as pltpu
from jax.experimental.pallas import tpu_sc as plsc

N, E, D, G = 10000, 320000, 128, 16
NC, NS = 2, 16
NW = NC * NS
NT = 10240
NSR = 10112
NCNT = NSR // 2
EPW = 10080
EPAD = NW * EPW
CHUNK = 48
NCHUNK = EPW // CHUNK
TW = D + 16
ROWS_PT = NSR // NS


def _tc_pre(hpad, pos16, w1a, w1b, be1):
    blk = 256

    def body(h_ref, p_ref, wa_ref, wb_ref, b_ref, ta_ref, tb_ref):
        hb = h_ref[...]
        a = jnp.dot(hb, wa_ref[...], preferred_element_type=jnp.float32)
        b = jnp.dot(hb, wb_ref[...], preferred_element_type=jnp.float32)
        p = p_ref[...]
        ta_ref[...] = jnp.concatenate([a, p], axis=1)
        tb_ref[...] = jnp.concatenate([b + b_ref[...], p], axis=1)

    return pl.pallas_call(
        body,
        grid=(NT // blk,),
        in_specs=[
            pl.BlockSpec((blk, D), lambda i: (i, 0)),
            pl.BlockSpec((blk, 16), lambda i: (i, 0)),
            pl.BlockSpec((D, D), lambda i: (0, 0)),
            pl.BlockSpec((D, D), lambda i: (0, 0)),
            pl.BlockSpec((1, D), lambda i: (0, 0)),
        ],
        out_specs=[
            pl.BlockSpec((blk, TW), lambda i: (i, 0)),
            pl.BlockSpec((blk, TW), lambda i: (i, 0)),
        ],
        out_shape=[
            jax.ShapeDtypeStruct((NT, TW), jnp.float32),
            jax.ShapeDtypeStruct((NT, TW), jnp.float32),
        ],
    )(hpad, pos16, w1a, w1b, be1)


def _sc_edge(ta, tb, srcp, dstp, wr, offsw, invw, zrows):
    mesh = plsc.VectorSubcoreMesh(core_axis_name="c", subcore_axis_name="s")
    f32 = jnp.float32

    @functools.partial(
        pl.kernel,
        mesh=mesh,
        compiler_params=pltpu.CompilerParams(
            needs_layout_passes=False, use_tc_tiling_on_sc=False),
        out_type=[
            jax.ShapeDtypeStruct((NC, NSR, D), jnp.float32),
            jax.ShapeDtypeStruct((NW * NCNT,), jnp.int32),
        ],
        scratch_types=[
            pltpu.VMEM((2, CHUNK), jnp.int32),
            pltpu.VMEM((2, CHUNK), jnp.int32),
            pltpu.VMEM((2, CHUNK), jnp.int32),
            pltpu.VMEM((2, CHUNK, TW), jnp.float32),
            pltpu.VMEM((2, CHUNK, TW), jnp.float32),
            pltpu.VMEM((2, CHUNK, D), jnp.float32),
            pltpu.VMEM((G, D), jnp.float32),
            pltpu.VMEM((16,), jnp.float32),
            pltpu.VMEM((16,), jnp.float32),
            pltpu.VMEM((NCNT,), jnp.int32),
            pltpu.VMEM_SHARED((NSR, D), jnp.float32),
            pltpu.SemaphoreType.DMA((2,)),
            pltpu.SemaphoreType.DMA((2,)),
            pltpu.SemaphoreType.DMA((2,)),
        ],
    )
    def k(ta_h, tb_h, src_h, dst_h, wr_h, offsw_h, invw_h, z_h,
          out_h, outc_h,
          sidx, didx, dsc, ga, gb, vbuf, wr_v, offs_v, invw_v, cnt_v, s_sh,
          sa, sb, ss):
        cid = lax.axis_index("c")
        sid = lax.axis_index("s")
        wid = sid * NC + cid

        pltpu.sync_copy(wr_h, wr_v)
        pltpu.sync_copy(offsw_h, offs_v)
        pltpu.sync_copy(invw_h, invw_v)

        def zc(i, _):
            cnt_v[pl.ds(16 * i, 16)] = jnp.zeros((16,), jnp.int32)
            return 0

        lax.fori_loop(0, NCNT // 16, zc, 0)

        pltpu.sync_copy(z_h, s_sh.at[pl.ds(sid * ROWS_PT, ROWS_PT)])
        plsc.subcore_barrier()

        lanes = lax.broadcasted_iota(jnp.int32, (16,), 0)
        one_i = jnp.full((16,), 1, jnp.int32)
        ebase = wid * EPW

        def fetch(j, p):
            pltpu.sync_copy(src_h.at[pl.ds(ebase + j * CHUNK, CHUNK)],
                            sidx.at[p])
            pltpu.sync_copy(dst_h.at[pl.ds(ebase + j * CHUNK, CHUNK)],
                            didx.at[p])
            pltpu.async_copy(ta_h.at[sidx.at[p]], ga.at[p], sa.at[p])
            pltpu.async_copy(tb_h.at[didx.at[p]], gb.at[p], sb.at[p])

        def wait_gathers(p):
            pltpu.make_async_copy(ta_h.at[sidx.at[p]], ga.at[p],
                                  sa.at[p]).wait()
            pltpu.make_async_copy(tb_h.at[didx.at[p]], gb.at[p],
                                  sb.at[p]).wait()

        def wait_scatter(p):
            pltpu.make_async_copy(vbuf.at[p], s_sh.at[dsc.at[p]],
                                  ss.at[p]).wait()

        def half(t2, p):
            j = 2 * t2 + p

            def prefetch():
                fetch(j + 1, 1 - p)

            if p == 0:
                prefetch()
            else:
                pl.when(t2 < NCHUNK // 2 - 1)(prefetch)

            wait_gathers(p)
            pl.when(t2 >= 1)(lambda: wait_scatter(p))

            for g2 in range(CHUNK // 16):
                dw = didx[p, pl.ds(16 * g2, 16)]
                idx2 = lax.shift_right_logical(dw, jnp.int32(1))
                val = one_i + (dw & 1) * 65535
                plsc.addupdate_scatter(cnt_v, [idx2], val)
                dsc[p, pl.ds(16 * g2, 16)] = dw

            def edge_body(e, __):
                dv = ga[p, e, pl.ds(D, 16)] - gb[p, e, pl.ds(D, 16)]
                s2 = dv * dv
                for sh in (8, 4, 2, 1):
                    s2 = s2 + s2.at[lanes ^ sh].get(
                        mode="promise_in_bounds")
                s2e = jnp.maximum(s2, 1e-30)
                ib = lax.bitcast_convert_type(s2e, jnp.int32)
                ib = 0x5F3759DF - lax.shift_right_logical(ib, jnp.int32(1))
                y = lax.bitcast_convert_type(ib, jnp.float32)
                y = y * (1.5 - 0.5 * s2e * y * y)
                y = y * (1.5 - 0.5 * s2e * y * y)
                y = y * (1.5 - 0.5 * s2e * y * y)
                dist = s2 * y
                t = dist * invw_v[...] - offs_v[...]
                f = jnp.exp(-0.5 * t * t)
                us = [ga[p, e, pl.ds(16 * q, 16)]
                      + gb[p, e, pl.ds(16 * q, 16)] for q in range(8)]
                for g in range(G):
                    fg = f.at[jnp.full((16,), g, jnp.int32)].get(
                        mode="promise_in_bounds")
                    for q in range(8):
                        us[q] = us[q] + fg * wr_v[g, pl.ds(16 * q, 16)]
                for q in range(8):
                    u = us[q]
                    vbuf[p, e, pl.ds(16 * q, 16)] = u / (1.0 + jnp.exp(-u))
                return 0

            lax.fori_loop(0, CHUNK, edge_body, 0, unroll=2)
            pltpu.async_copy(vbuf.at[p], s_sh.at[dsc.at[p]], ss.at[p],
                             add=True)

        fetch(0, 0)

        def pair_body(t2, _):
            half(t2, 0)
            half(t2, 1)
            return 0

        lax.fori_loop(0, NCHUNK // 2, pair_body, 0)
        wait_scatter(0)
        wait_scatter(1)

        pltpu.sync_copy(cnt_v, outc_h.at[pl.ds(wid * NCNT, NCNT)])
        plsc.subcore_barrier()
        pltpu.sync_copy(
            s_sh.at[pl.ds(sid * ROWS_PT, ROWS_PT)],
            out_h.at[cid, pl.ds(sid * ROWS_PT, ROWS_PT)],
        )

    return k(ta, tb, srcp, dstp, wr, offsw, invw, zrows)


def _tc_post(h, s0, s1, cntT, we2, be2, wn1, bn1, wn2, bn2, wsc, bsc,
             gamma, beta):
    blk = 400

    def body(h_ref, s0_ref, s1_ref, c_ref, we2_ref, be2_ref, wn1_ref,
             bn1_ref, wn2_ref, bn2_ref, wsc_ref, bsc_ref, g_ref, be_ref,
             o_ref):
        s = s0_ref[...] + s1_ref[...]
        cnt = jnp.sum(c_ref[...], axis=1, keepdims=True)
        agg = (jnp.dot(s, we2_ref[...], preferred_element_type=jnp.float32)
               + cnt * be2_ref[...])
        x = jnp.concatenate([h_ref[...], agg], axis=-1)
        t1 = jnp.dot(x, wn1_ref[...], preferred_element_type=jnp.float32)
        t1 = t1 + bn1_ref[...]
        t1 = t1 * (1.0 / (1.0 + jnp.exp(-t1)))
        y = (jnp.dot(t1, wn2_ref[...], preferred_element_type=jnp.float32)
             + bn2_ref[...]
             + jnp.dot(x, wsc_ref[...], preferred_element_type=jnp.float32)
             + bsc_ref[...])
        mu = jnp.mean(y, axis=-1, keepdims=True)
        yc = y - mu
        var = jnp.mean(yc * yc, axis=-1, keepdims=True)
        o_ref[...] = yc * lax.rsqrt(var + 1e-5) * g_ref[...] + be_ref[...]

    full = lambda shape: pl.BlockSpec(shape, lambda i: (0, 0))
    return pl.pallas_call(
        body,
        grid=(N // blk,),
        in_specs=[
            pl.BlockSpec((blk, D), lambda i: (i, 0)),
            pl.BlockSpec((blk, D), lambda i: (i, 0)),
            pl.BlockSpec((blk, D), lambda i: (i, 0)),
            pl.BlockSpec((blk, NW), lambda i: (i, 0)),
            full((D, D)),
            full((1, D)),
            full((2 * D, D)),
            full((1, D)),
            full((D, D)),
            full((1, D)),
            full((2 * D, D)),
            full((1, D)),
            full((1, D)),
            full((1, D)),
        ],
        out_specs=pl.BlockSpec((blk, D), lambda i: (i, 0)),
        out_shape=jax.ShapeDtypeStruct((N, D), jnp.float32),
    )(h, s0, s1, cntT, we2, be2, wn1, bn1, wn2, bn2, wsc, bsc, gamma, beta)


def kernel(h, pos, edge_index, W_e1, b_e1, W_e2, b_e2, W_n1, b_n1, W_n2,
           b_n2, W_sc, b_sc, gamma, beta, offsets):
    f32 = jnp.float32
    w1a = W_e1[:D]
    w1b = W_e1[D:2 * D]
    wr = W_e1[2 * D:]
    width = offsets[1] - offsets[0]
    offsw = (offsets / width).astype(f32)
    invw = jnp.full((16,), 1.0, f32) / width

    hpad = jnp.zeros((NT, D), f32).at[:N].set(h)
    pos16 = jnp.zeros((NT, 16), f32).at[:N, :3].set(pos)
    ta, tb = _tc_pre(hpad, pos16, w1a, w1b, b_e1.reshape(1, D))

    pad_idx = jnp.full((EPAD - E,), N, jnp.int32)
    srcp = jnp.concatenate([edge_index[0], pad_idx])
    dstp = jnp.concatenate([edge_index[1], pad_idx])
    zrows = jnp.zeros((ROWS_PT, D), f32)
    part, outc = _sc_edge(ta, tb, srcp, dstp, wr, offsw, invw, zrows)

    c2 = outc.reshape(NW, NCNT)
    counts = jnp.stack([c2 & 0xFFFF, lax.shift_right_logical(c2, 16)],
                       axis=-1).reshape(NW, NSR)
    cntT = counts.T[:N].astype(f32)
    h_new = _tc_post(h, part[0, :N], part[1, :N], cntT, W_e2,
                     b_e2.reshape(1, D), W_n1, b_n1.reshape(1, D), W_n2,
                     b_n2.reshape(1, D), W_sc, b_sc.reshape(1, D),
                     gamma.reshape(1, D), beta.reshape(1, D))
    return (h_new, pos)


# speedup vs baseline: 1.5592x; 1.0078x over previous
import functools

import jax
import jax.numpy as jnp
from jax import lax
from jax.experimental import pallas as pl
from jax.experimental.pallas import tpu as pltpu
from jax.experimental.pallas import tpu_sc as plsc

N, E, D, G = 10000, 320000, 128, 16
NC, NS = 2, 16
NW = NC * NS
NT = 10240
NSR = 10112
NCNT = NSR // 2
EPW = 10080
EPAD = NW * EPW
CHUNK = 48
NCHUNK = EPW // CHUNK
TW = D + 16
ROWS_PT = NSR // NS


def _tc_pre(hpad, pos16, w1a, w1b, be1):
    blk = 256

    def body(h_ref, p_ref, wa_ref, wb_ref, b_ref, ta_ref, tb_ref):
        hb = h_ref[...]
        a = jnp.dot(hb, wa_ref[...], preferred_element_type=jnp.float32)
        b = jnp.dot(hb, wb_ref[...], preferred_element_type=jnp.float32)
        p = p_ref[...]
        ta_ref[...] = jnp.concatenate([a, p], axis=1)
        tb_ref[...] = jnp.concatenate([b + b_ref[...], p], axis=1)

    return pl.pallas_call(
        body,
        grid=(NT // blk,),
        in_specs=[
            pl.BlockSpec((blk, D), lambda i: (i, 0)),
            pl.BlockSpec((blk, 16), lambda i: (i, 0)),
            pl.BlockSpec((D, D), lambda i: (0, 0)),
            pl.BlockSpec((D, D), lambda i: (0, 0)),
            pl.BlockSpec((1, D), lambda i: (0, 0)),
        ],
        out_specs=[
            pl.BlockSpec((blk, TW), lambda i: (i, 0)),
            pl.BlockSpec((blk, TW), lambda i: (i, 0)),
        ],
        out_shape=[
            jax.ShapeDtypeStruct((NT, TW), jnp.float32),
            jax.ShapeDtypeStruct((NT, TW), jnp.float32),
        ],
    )(hpad, pos16, w1a, w1b, be1)


def _sc_edge(ta, tb, srcp, dstp, wr, offsw, invw, zrows):
    mesh = plsc.VectorSubcoreMesh(core_axis_name="c", subcore_axis_name="s")
    f32 = jnp.float32

    @functools.partial(
        pl.kernel,
        mesh=mesh,
        compiler_params=pltpu.CompilerParams(
            needs_layout_passes=False, use_tc_tiling_on_sc=False),
        out_type=[
            jax.ShapeDtypeStruct((NC, NSR, D), jnp.float32),
            jax.ShapeDtypeStruct((NW * NCNT,), jnp.int32),
        ],
        scratch_types=[
            pltpu.VMEM((2, CHUNK), jnp.int32),
            pltpu.VMEM((2, CHUNK), jnp.int32),
            pltpu.VMEM((2, CHUNK), jnp.int32),
            pltpu.VMEM((2, CHUNK, TW), jnp.float32),
            pltpu.VMEM((2, CHUNK, TW), jnp.float32),
            pltpu.VMEM((2, CHUNK, D), jnp.float32),
            pltpu.VMEM((G, D), jnp.float32),
            pltpu.VMEM((16,), jnp.float32),
            pltpu.VMEM((16,), jnp.float32),
            pltpu.VMEM((NCNT,), jnp.int32),
            pltpu.VMEM_SHARED((NSR, D), jnp.float32),
            pltpu.SemaphoreType.DMA((2,)),
            pltpu.SemaphoreType.DMA((2,)),
            pltpu.SemaphoreType.DMA((2,)),
        ],
    )
    def k(ta_h, tb_h, src_h, dst_h, wr_h, offsw_h, invw_h, z_h,
          out_h, outc_h,
          sidx, didx, dsc, ga, gb, vbuf, wr_v, offs_v, invw_v, cnt_v, s_sh,
          sa, sb, ss):
        cid = lax.axis_index("c")
        sid = lax.axis_index("s")
        wid = sid * NC + cid

        pltpu.sync_copy(wr_h, wr_v)
        pltpu.sync_copy(offsw_h, offs_v)
        pltpu.sync_copy(invw_h, invw_v)

        def zc(i, _):
            cnt_v[pl.ds(16 * i, 16)] = jnp.zeros((16,), jnp.int32)
            return 0

        lax.fori_loop(0, NCNT // 16, zc, 0)

        pltpu.sync_copy(z_h, s_sh.at[pl.ds(sid * ROWS_PT, ROWS_PT)])
        plsc.subcore_barrier()

        lanes = lax.broadcasted_iota(jnp.int32, (16,), 0)
        one_i = jnp.full((16,), 1, jnp.int32)
        ebase = wid * EPW

        def fetch(j, p):
            pltpu.sync_copy(src_h.at[pl.ds(ebase + j * CHUNK, CHUNK)],
                            sidx.at[p])
            pltpu.sync_copy(dst_h.at[pl.ds(ebase + j * CHUNK, CHUNK)],
                            didx.at[p])
            pltpu.async_copy(ta_h.at[sidx.at[p]], ga.at[p], sa.at[p])
            pltpu.async_copy(tb_h.at[didx.at[p]], gb.at[p], sb.at[p])

        def wait_gathers(p):
            pltpu.make_async_copy(ta_h.at[sidx.at[p]], ga.at[p],
                                  sa.at[p]).wait()
            pltpu.make_async_copy(tb_h.at[didx.at[p]], gb.at[p],
                                  sb.at[p]).wait()

        def wait_scatter(p):
            pltpu.make_async_copy(vbuf.at[p], s_sh.at[dsc.at[p]],
                                  ss.at[p]).wait()

        def half(t2, p):
            j = 2 * t2 + p

            def prefetch():
                fetch(j + 1, 1 - p)

            if p == 0:
                prefetch()
            else:
                pl.when(t2 < NCHUNK // 2 - 1)(prefetch)

            wait_gathers(p)
            pl.when(t2 >= 1)(lambda: wait_scatter(p))

            for g2 in range(CHUNK // 16):
                dw = didx[p, pl.ds(16 * g2, 16)]
                idx2 = lax.shift_right_logical(dw, jnp.int32(1))
                val = one_i + (dw & 1) * 65535
                plsc.addupdate_scatter(cnt_v, [idx2], val)
                dsc[p, pl.ds(16 * g2, 16)] = dw

            def edge_body(e, __):
                dv = ga[p, e, pl.ds(D, 16)] - gb[p, e, pl.ds(D, 16)]
                s2 = dv * dv
                for sh in (8, 4, 2, 1):
                    s2 = s2 + s2.at[lanes ^ sh].get(
                        mode="promise_in_bounds")
                s2e = jnp.maximum(s2, 1e-30)
                ib = lax.bitcast_convert_type(s2e, jnp.int32)
                ib = 0x5F3759DF - lax.shift_right_logical(ib, jnp.int32(1))
                y = lax.bitcast_convert_type(ib, jnp.float32)
                y = y * (1.5 - 0.5 * s2e * y * y)
                y = y * (1.5 - 0.5 * s2e * y * y)
                dist = s2 * y
                t = dist * invw_v[...] - offs_v[...]
                f = jnp.exp(-0.5 * t * t)
                us = [ga[p, e, pl.ds(16 * q, 16)]
                      + gb[p, e, pl.ds(16 * q, 16)] for q in range(8)]
                for g in range(G):
                    fg = f.at[jnp.full((16,), g, jnp.int32)].get(
                        mode="promise_in_bounds")
                    for q in range(8):
                        us[q] = us[q] + fg * wr_v[g, pl.ds(16 * q, 16)]
                for q in range(8):
                    u = us[q]
                    vbuf[p, e, pl.ds(16 * q, 16)] = u / (1.0 + jnp.exp(-u))
                return 0

            lax.fori_loop(0, CHUNK, edge_body, 0, unroll=4)
            pltpu.async_copy(vbuf.at[p], s_sh.at[dsc.at[p]], ss.at[p],
                             add=True)

        fetch(0, 0)

        def pair_body(t2, _):
            half(t2, 0)
            half(t2, 1)
            return 0

        lax.fori_loop(0, NCHUNK // 2, pair_body, 0)
        wait_scatter(0)
        wait_scatter(1)

        pltpu.sync_copy(cnt_v, outc_h.at[pl.ds(wid * NCNT, NCNT)])
        plsc.subcore_barrier()
        pltpu.sync_copy(
            s_sh.at[pl.ds(sid * ROWS_PT, ROWS_PT)],
            out_h.at[cid, pl.ds(sid * ROWS_PT, ROWS_PT)],
        )

    return k(ta, tb, srcp, dstp, wr, offsw, invw, zrows)


def _tc_post(h, s0, s1, cntT, we2, be2, wn1, bn1, wn2, bn2, wsc, bsc,
             gamma, beta):
    blk = 400

    def body(h_ref, s0_ref, s1_ref, c_ref, we2_ref, be2_ref, wn1_ref,
             bn1_ref, wn2_ref, bn2_ref, wsc_ref, bsc_ref, g_ref, be_ref,
             o_ref):
        s = s0_ref[...] + s1_ref[...]
        cnt = jnp.sum(c_ref[...], axis=1, keepdims=True)
        agg = (jnp.dot(s, we2_ref[...], preferred_element_type=jnp.float32)
               + cnt * be2_ref[...])
        x = jnp.concatenate([h_ref[...], agg], axis=-1)
        t1 = jnp.dot(x, wn1_ref[...], preferred_element_type=jnp.float32)
        t1 = t1 + bn1_ref[...]
        t1 = t1 * (1.0 / (1.0 + jnp.exp(-t1)))
        y = (jnp.dot(t1, wn2_ref[...], preferred_element_type=jnp.float32)
             + bn2_ref[...]
             + jnp.dot(x, wsc_ref[...], preferred_element_type=jnp.float32)
             + bsc_ref[...])
        mu = jnp.mean(y, axis=-1, keepdims=True)
        yc = y - mu
        var = jnp.mean(yc * yc, axis=-1, keepdims=True)
        o_ref[...] = yc * lax.rsqrt(var + 1e-5) * g_ref[...] + be_ref[...]

    full = lambda shape: pl.BlockSpec(shape, lambda i: (0, 0))
    return pl.pallas_call(
        body,
        grid=(N // blk,),
        in_specs=[
            pl.BlockSpec((blk, D), lambda i: (i, 0)),
            pl.BlockSpec((blk, D), lambda i: (i, 0)),
            pl.BlockSpec((blk, D), lambda i: (i, 0)),
            pl.BlockSpec((blk, NW), lambda i: (i, 0)),
            full((D, D)),
            full((1, D)),
            full((2 * D, D)),
            full((1, D)),
            full((D, D)),
            full((1, D)),
            full((2 * D, D)),
            full((1, D)),
            full((1, D)),
            full((1, D)),
        ],
        out_specs=pl.BlockSpec((blk, D), lambda i: (i, 0)),
        out_shape=jax.ShapeDtypeStruct((N, D), jnp.float32),
    )(h, s0, s1, cntT, we2, be2, wn1, bn1, wn2, bn2, wsc, bsc, gamma, beta)


def kernel(h, pos, edge_index, W_e1, b_e1, W_e2, b_e2, W_n1, b_n1, W_n2,
           b_n2, W_sc, b_sc, gamma, beta, offsets):
    f32 = jnp.float32
    w1a = W_e1[:D]
    w1b = W_e1[D:2 * D]
    wr = W_e1[2 * D:]
    width = offsets[1] - offsets[0]
    offsw = (offsets / width).astype(f32)
    invw = jnp.full((16,), 1.0, f32) / width

    hpad = jnp.zeros((NT, D), f32).at[:N].set(h)
    pos16 = jnp.zeros((NT, 16), f32).at[:N, :3].set(pos)
    ta, tb = _tc_pre(hpad, pos16, w1a, w1b, b_e1.reshape(1, D))

    pad_idx = jnp.full((EPAD - E,), N, jnp.int32)
    srcp = jnp.concatenate([edge_index[0], pad_idx])
    dstp = jnp.concatenate([edge_index[1], pad_idx])
    zrows = jnp.zeros((ROWS_PT, D), f32)
    part, outc = _sc_edge(ta, tb, srcp, dstp, wr, offsw, invw, zrows)

    c2 = outc.reshape(NW, NCNT)
    counts = jnp.stack([c2 & 0xFFFF, lax.shift_right_logical(c2, 16)],
                       axis=-1).reshape(NW, NSR)
    cntT = counts.T[:N].astype(f32)
    h_new = _tc_post(h, part[0, :N], part[1, :N], cntT, W_e2,
                     b_e2.reshape(1, D), W_n1, b_n1.reshape(1, D), W_n2,
                     b_n2.reshape(1, D), W_sc, b_sc.reshape(1, D),
                     gamma.reshape(1, D), beta.reshape(1, D))
    return (h_new, pos)


# speedup vs baseline: 1.7417x; 1.1170x over previous
import functools

import jax
import jax.numpy as jnp
from jax import lax
from jax.experimental import pallas as pl
from jax.experimental.pallas import tpu as pltpu
from jax.experimental.pallas import tpu_sc as plsc

N, E, D, G = 10000, 320000, 128, 16
NC, NS = 2, 16
NW = NC * NS
NT = 10240
NSR = 10112
NCNT = NSR // 2
EPW = 10080
EPAD = NW * EPW
CHUNK = 48
NCHUNK = EPW // CHUNK
TW = D + 16
ROWS_PT = NSR // NS


def _tc_pre(hpad, pos16, w1a, w1b, be1):
    blk = 256

    def body(h_ref, p_ref, wa_ref, wb_ref, b_ref, ta_ref, tb_ref):
        hb = h_ref[...]
        a = jnp.dot(hb, wa_ref[...], preferred_element_type=jnp.float32)
        b = jnp.dot(hb, wb_ref[...], preferred_element_type=jnp.float32)
        p = p_ref[...]
        ta_ref[...] = jnp.concatenate([a, p], axis=1)
        tb_ref[...] = jnp.concatenate([b + b_ref[...], p], axis=1)

    return pl.pallas_call(
        body,
        grid=(NT // blk,),
        in_specs=[
            pl.BlockSpec((blk, D), lambda i: (i, 0)),
            pl.BlockSpec((blk, 16), lambda i: (i, 0)),
            pl.BlockSpec((D, D), lambda i: (0, 0)),
            pl.BlockSpec((D, D), lambda i: (0, 0)),
            pl.BlockSpec((1, D), lambda i: (0, 0)),
        ],
        out_specs=[
            pl.BlockSpec((blk, TW), lambda i: (i, 0)),
            pl.BlockSpec((blk, TW), lambda i: (i, 0)),
        ],
        out_shape=[
            jax.ShapeDtypeStruct((NT, TW), jnp.float32),
            jax.ShapeDtypeStruct((NT, TW), jnp.float32),
        ],
    )(hpad, pos16, w1a, w1b, be1)


def _sc_edge(ta, tb, srcp, dstp, wr, offsw, invw, zrows):
    mesh = plsc.VectorSubcoreMesh(core_axis_name="c", subcore_axis_name="s")
    f32 = jnp.float32

    @functools.partial(
        pl.kernel,
        mesh=mesh,
        compiler_params=pltpu.CompilerParams(
            needs_layout_passes=False, use_tc_tiling_on_sc=False),
        out_type=[
            jax.ShapeDtypeStruct((NC, NSR, D), jnp.float32),
            jax.ShapeDtypeStruct((NW * NCNT,), jnp.int32),
        ],
        scratch_types=[
            pltpu.VMEM((2, CHUNK), jnp.int32),
            pltpu.VMEM((2, CHUNK), jnp.int32),
            pltpu.VMEM((2, CHUNK), jnp.int32),
            pltpu.VMEM((2, CHUNK, TW), jnp.float32),
            pltpu.VMEM((2, CHUNK, TW), jnp.float32),
            pltpu.VMEM((2, CHUNK, D), jnp.float32),
            pltpu.VMEM((G, D), jnp.float32),
            pltpu.VMEM((16,), jnp.float32),
            pltpu.VMEM((16,), jnp.float32),
            pltpu.VMEM((NCNT,), jnp.int32),
            pltpu.VMEM_SHARED((NSR, D), jnp.float32),
            pltpu.SemaphoreType.DMA((2,)),
            pltpu.SemaphoreType.DMA((2,)),
            pltpu.SemaphoreType.DMA((2,)),
            pltpu.SemaphoreType.DMA((2,)),
            pltpu.SemaphoreType.DMA((2,)),
        ],
    )
    def k(ta_h, tb_h, src_h, dst_h, wr_h, offsw_h, invw_h, z_h,
          out_h, outc_h,
          sidx, didx, dsc, ga, gb, vbuf, wr_v, offs_v, invw_v, cnt_v, s_sh,
          sa, sb, ss, sis, sid_s):
        cid = lax.axis_index("c")
        sid = lax.axis_index("s")
        wid = sid * NC + cid

        pltpu.sync_copy(wr_h, wr_v)
        pltpu.sync_copy(offsw_h, offs_v)
        pltpu.sync_copy(invw_h, invw_v)

        def zc(i, _):
            cnt_v[pl.ds(16 * i, 16)] = jnp.zeros((16,), jnp.int32)
            return 0

        lax.fori_loop(0, NCNT // 16, zc, 0)

        pltpu.sync_copy(z_h, s_sh.at[pl.ds(sid * ROWS_PT, ROWS_PT)])
        plsc.subcore_barrier()

        lanes = lax.broadcasted_iota(jnp.int32, (16,), 0)
        one_i = jnp.full((16,), 1, jnp.int32)
        ebase = wid * EPW

        def idx_copies(j, p):
            base = ebase + j * CHUNK
            return (
                pltpu.make_async_copy(src_h.at[pl.ds(base, CHUNK)],
                                      sidx.at[p], sis.at[p]),
                pltpu.make_async_copy(dst_h.at[pl.ds(base, CHUNK)],
                                      didx.at[p], sid_s.at[p]),
            )

        def fetch_idx(j, p):
            c1, c2 = idx_copies(j, p)
            c1.start()
            c2.start()

        def wait_idx(j, p):
            c1, c2 = idx_copies(j, p)
            c1.wait()
            c2.wait()

        def gathers(p):
            return (
                pltpu.make_async_copy(ta_h.at[sidx.at[p]], ga.at[p],
                                      sa.at[p]),
                pltpu.make_async_copy(tb_h.at[didx.at[p]], gb.at[p],
                                      sb.at[p]),
            )

        def fetch_gathers(p):
            g1, g2 = gathers(p)
            g1.start()
            g2.start()

        def wait_gathers(p):
            g1, g2 = gathers(p)
            g1.wait()
            g2.wait()

        def wait_scatter(p):
            pltpu.make_async_copy(vbuf.at[p], s_sh.at[dsc.at[p]],
                                  ss.at[p]).wait()

        def half(t2, p):
            j = 2 * t2 + p
            wait_gathers(p)
            pl.when(t2 >= 1)(lambda: wait_scatter(p))

            for g2 in range(CHUNK // 16):
                dw = didx[p, pl.ds(16 * g2, 16)]
                idx2 = lax.shift_right_logical(dw, jnp.int32(1))
                val = one_i + (dw & 1) * 65535
                plsc.addupdate_scatter(cnt_v, [idx2], val)
                dsc[p, pl.ds(16 * g2, 16)] = dw

            pl.when(t2 < NCHUNK // 2 - 1)(lambda: fetch_idx(j + 2, p))

            def launch_next():
                wait_idx(j + 1, 1 - p)
                fetch_gathers(1 - p)

            if p == 0:
                launch_next()
            else:
                pl.when(t2 < NCHUNK // 2 - 1)(launch_next)

            def edge_body(e, __):
                dv = ga[p, e, pl.ds(D, 16)] - gb[p, e, pl.ds(D, 16)]
                s2 = dv * dv
                for sh in (8, 4, 2, 1):
                    s2 = s2 + s2.at[lanes ^ sh].get(
                        mode="promise_in_bounds")
                s2e = jnp.maximum(s2, 1e-30)
                ib = lax.bitcast_convert_type(s2e, jnp.int32)
                ib = 0x5F3759DF - lax.shift_right_logical(ib, jnp.int32(1))
                y = lax.bitcast_convert_type(ib, jnp.float32)
                y = y * (1.5 - 0.5 * s2e * y * y)
                y = y * (1.5 - 0.5 * s2e * y * y)
                dist = s2 * y
                t = dist * invw_v[...] - offs_v[...]
                f = jnp.exp(-0.5 * t * t)
                us = [ga[p, e, pl.ds(16 * q, 16)]
                      + gb[p, e, pl.ds(16 * q, 16)] for q in range(8)]
                for g in range(G):
                    fg = f.at[jnp.full((16,), g, jnp.int32)].get(
                        mode="promise_in_bounds")
                    for q in range(8):
                        us[q] = us[q] + fg * wr_v[g, pl.ds(16 * q, 16)]
                for q in range(8):
                    u = us[q]
                    vbuf[p, e, pl.ds(16 * q, 16)] = u / (1.0 + jnp.exp(-u))
                return 0

            lax.fori_loop(0, CHUNK, edge_body, 0, unroll=4)
            pltpu.async_copy(vbuf.at[p], s_sh.at[dsc.at[p]], ss.at[p],
                             add=True)

        fetch_idx(0, 0)
        wait_idx(0, 0)
        fetch_gathers(0)
        fetch_idx(1, 1)

        def pair_body(t2, _):
            half(t2, 0)
            half(t2, 1)
            return 0

        lax.fori_loop(0, NCHUNK // 2, pair_body, 0)
        wait_scatter(0)
        wait_scatter(1)

        pltpu.sync_copy(cnt_v, outc_h.at[pl.ds(wid * NCNT, NCNT)])
        plsc.subcore_barrier()
        pltpu.sync_copy(
            s_sh.at[pl.ds(sid * ROWS_PT, ROWS_PT)],
            out_h.at[cid, pl.ds(sid * ROWS_PT, ROWS_PT)],
        )

    return k(ta, tb, srcp, dstp, wr, offsw, invw, zrows)


def _tc_post(h, s0, s1, cntT, we2, be2, wn1, bn1, wn2, bn2, wsc, bsc,
             gamma, beta):
    blk = 400

    def body(h_ref, s0_ref, s1_ref, c_ref, we2_ref, be2_ref, wn1_ref,
             bn1_ref, wn2_ref, bn2_ref, wsc_ref, bsc_ref, g_ref, be_ref,
             o_ref):
        s = s0_ref[...] + s1_ref[...]
        cnt = jnp.sum(c_ref[...], axis=1, keepdims=True)
        agg = (jnp.dot(s, we2_ref[...], preferred_element_type=jnp.float32)
               + cnt * be2_ref[...])
        x = jnp.concatenate([h_ref[...], agg], axis=-1)
        t1 = jnp.dot(x, wn1_ref[...], preferred_element_type=jnp.float32)
        t1 = t1 + bn1_ref[...]
        t1 = t1 * (1.0 / (1.0 + jnp.exp(-t1)))
        y = (jnp.dot(t1, wn2_ref[...], preferred_element_type=jnp.float32)
             + bn2_ref[...]
             + jnp.dot(x, wsc_ref[...], preferred_element_type=jnp.float32)
             + bsc_ref[...])
        mu = jnp.mean(y, axis=-1, keepdims=True)
        yc = y - mu
        var = jnp.mean(yc * yc, axis=-1, keepdims=True)
        o_ref[...] = yc * lax.rsqrt(var + 1e-5) * g_ref[...] + be_ref[...]

    full = lambda shape: pl.BlockSpec(shape, lambda i: (0, 0))
    return pl.pallas_call(
        body,
        grid=(N // blk,),
        in_specs=[
            pl.BlockSpec((blk, D), lambda i: (i, 0)),
            pl.BlockSpec((blk, D), lambda i: (i, 0)),
            pl.BlockSpec((blk, D), lambda i: (i, 0)),
            pl.BlockSpec((blk, NW), lambda i: (i, 0)),
            full((D, D)),
            full((1, D)),
            full((2 * D, D)),
            full((1, D)),
            full((D, D)),
            full((1, D)),
            full((2 * D, D)),
            full((1, D)),
            full((1, D)),
            full((1, D)),
        ],
        out_specs=pl.BlockSpec((blk, D), lambda i: (i, 0)),
        out_shape=jax.ShapeDtypeStruct((N, D), jnp.float32),
    )(h, s0, s1, cntT, we2, be2, wn1, bn1, wn2, bn2, wsc, bsc, gamma, beta)


def kernel(h, pos, edge_index, W_e1, b_e1, W_e2, b_e2, W_n1, b_n1, W_n2,
           b_n2, W_sc, b_sc, gamma, beta, offsets):
    f32 = jnp.float32
    w1a = W_e1[:D]
    w1b = W_e1[D:2 * D]
    wr = W_e1[2 * D:]
    width = offsets[1] - offsets[0]
    offsw = (offsets / width).astype(f32)
    invw = jnp.full((16,), 1.0, f32) / width

    hpad = jnp.zeros((NT, D), f32).at[:N].set(h)
    pos16 = jnp.zeros((NT, 16), f32).at[:N, :3].set(pos)
    ta, tb = _tc_pre(hpad, pos16, w1a, w1b, b_e1.reshape(1, D))

    pad_idx = jnp.full((EPAD - E,), N, jnp.int32)
    srcp = jnp.concatenate([edge_index[0], pad_idx])
    dstp = jnp.concatenate([edge_index[1], pad_idx])
    zrows = jnp.zeros((ROWS_PT, D), f32)
    part, outc = _sc_edge(ta, tb, srcp, dstp, wr, offsw, invw, zrows)

    c2 = outc.reshape(NW, NCNT)
    counts = jnp.stack([c2 & 0xFFFF, lax.shift_right_logical(c2, 16)],
                       axis=-1).reshape(NW, NSR)
    cntT = counts.T[:N].astype(f32)
    h_new = _tc_post(h, part[0, :N], part[1, :N], cntT, W_e2,
                     b_e2.reshape(1, D), W_n1, b_n1.reshape(1, D), W_n2,
                     b_n2.reshape(1, D), W_sc, b_sc.reshape(1, D),
                     gamma.reshape(1, D), beta.reshape(1, D))
    return (h_new, pos)
